# Initial kernel scaffold; baseline (speedup 1.0000x reference)
import functools

import jax
import jax.numpy as jnp
from jax import lax
from jax.experimental import pallas as pl
from jax.experimental.pallas import tpu as pltpu
from jax.experimental.pallas import tpu_sc as plsc

_BATCH = 16384
_HIST = 50
_DIM = 32
_B = _BATCH * _HIST
_NC = 2
_NS = 16
_NW = _NC * _NS
_BPW = _B // _NW
_CHUNK = 2560
_NCHUNK = _BPW // _CHUNK


@functools.partial(
    pl.kernel,
    out_type=jax.ShapeDtypeStruct((_B, _DIM), jnp.float32),
    mesh=plsc.VectorSubcoreMesh(core_axis_name="c", subcore_axis_name="s"),
    scratch_types=[
        pltpu.VMEM((_CHUNK,), jnp.int32),
        pltpu.VMEM((_CHUNK, _DIM), jnp.float32),
        pltpu.SemaphoreType.DMA,
    ],
)
def _sc_gather(table_hbm, idx_hbm, out_hbm, idx_v, rows_v, sem):
  wid = lax.axis_index("s") * _NC + lax.axis_index("c")
  base0 = wid * _BPW

  def body(g, carry):
    base = base0 + g * _CHUNK
    pltpu.sync_copy(idx_hbm.at[pl.ds(base, _CHUNK)], idx_v)
    pltpu.async_copy(table_hbm.at[idx_v], rows_v, sem).wait()
    pltpu.sync_copy(rows_v, out_hbm.at[pl.ds(base, _CHUNK)])
    return carry

  lax.fori_loop(0, _NCHUNK, body, 0)


def kernel(inputs, embeddings):
  idx = inputs.astype(jnp.int32).reshape(_B)
  out = _sc_gather(embeddings, idx)
  return out.reshape(_BATCH, _HIST, _DIM)


# speedup vs baseline: 1.1080x; 1.1080x over previous
import functools

import jax
import jax.numpy as jnp
from jax import lax
from jax.experimental import pallas as pl
from jax.experimental.pallas import tpu as pltpu
from jax.experimental.pallas import tpu_sc as plsc

_BATCH = 16384
_HIST = 50
_DIM = 32
_B = _BATCH * _HIST
_NC = 2
_NS = 16
_NW = _NC * _NS
_BPW = _B // _NW
_CHUNK = 2560
_NCHUNK = _BPW // _CHUNK


@functools.partial(
    pl.kernel,
    out_type=jax.ShapeDtypeStruct((_B, _DIM), jnp.float32),
    mesh=plsc.VectorSubcoreMesh(core_axis_name="c", subcore_axis_name="s"),
    scratch_types=[
        pltpu.VMEM((_CHUNK,), jnp.int32),
        pltpu.VMEM((_CHUNK, _DIM), jnp.float32),
        pltpu.SemaphoreType.DMA,
    ],
    compiler_params=pltpu.CompilerParams(use_tc_tiling_on_sc=False),
)
def _sc_gather(table_hbm, idx_hbm, out_hbm, idx_v, rows_v, sem):
  wid = lax.axis_index("s") * _NC + lax.axis_index("c")
  base0 = wid * _BPW

  def body(g, carry):
    base = base0 + g * _CHUNK
    pltpu.sync_copy(idx_hbm.at[pl.ds(base, _CHUNK)], idx_v)
    pltpu.async_copy(table_hbm.at[idx_v], rows_v, sem).wait()
    pltpu.sync_copy(rows_v, out_hbm.at[pl.ds(base, _CHUNK)])
    return carry

  lax.fori_loop(0, _NCHUNK, body, 0)


def kernel(inputs, embeddings):
  idx = inputs.astype(jnp.int32).reshape(_B)
  out = _sc_gather(embeddings, idx)
  return out.reshape(_BATCH, _HIST, _DIM)


# speedup vs baseline: 1.1120x; 1.0036x over previous
import functools

import jax
import jax.numpy as jnp
from jax import lax
from jax.experimental import pallas as pl
from jax.experimental.pallas import tpu as pltpu
from jax.experimental.pallas import tpu_sc as plsc

_BATCH = 16384
_HIST = 50
_DIM = 32
_B = _BATCH * _HIST
_NC = 2
_NS = 16
_NW = _NC * _NS
_BPW = _B // _NW
_C = 1024
_N = _BPW // _C
_NBUF = 3


@functools.partial(
    pl.kernel,
    out_type=jax.ShapeDtypeStruct((_B, _DIM), jnp.float32),
    mesh=plsc.VectorSubcoreMesh(core_axis_name="c", subcore_axis_name="s"),
    scratch_types=(
        [pltpu.VMEM((_BPW,), jnp.int32)]
        + [pltpu.VMEM((_C, _DIM), jnp.float32) for _ in range(_NBUF)]
        + [pltpu.SemaphoreType.DMA for _ in range(2 * _NBUF)]
    ),
    compiler_params=pltpu.CompilerParams(use_tc_tiling_on_sc=False),
)
def _sc_gather(table_hbm, idx_hbm, out_hbm, idx_v,
               r0, r1, r2, g0, g1, g2, s0, s1, s2):
  rows = (r0, r1, r2)
  gsem = (g0, g1, g2)
  ssem = (s0, s1, s2)
  wid = lax.axis_index("s") * _NC + lax.axis_index("c")
  base0 = wid * _BPW

  pltpu.sync_copy(idx_hbm.at[pl.ds(base0, _BPW)], idx_v)

  gcp = [None] * _NBUF
  scp = [None] * _NBUF
  depth = _NBUF - 1
  for t in range(_N + depth):
    if t < _N:
      b = t % _NBUF
      if t >= _NBUF:
        scp[b].wait()
      gcp[b] = pltpu.async_copy(
          table_hbm.at[idx_v.at[pl.ds(t * _C, _C)]], rows[b], gsem[b])
    u = t - depth
    if u >= 0:
      bu = u % _NBUF
      gcp[bu].wait()
      scp[bu] = pltpu.async_copy(
          rows[bu], out_hbm.at[pl.ds(base0 + u * _C, _C)], ssem[bu])
  for u in range(max(0, _N - _NBUF), _N):
    scp[u % _NBUF].wait()


def kernel(inputs, embeddings):
  idx = inputs.astype(jnp.int32).reshape(_B)
  out = _sc_gather(embeddings, idx)
  return out.reshape(_BATCH, _HIST, _DIM)


# speedup vs baseline: 1.3475x; 1.2119x over previous
import functools

import jax
import jax.numpy as jnp
from jax import lax
from jax.experimental import pallas as pl
from jax.experimental.pallas import tpu as pltpu
from jax.experimental.pallas import tpu_sc as plsc

_VOCAB = 1000000
_B = 16384
_H = 50
_D = 32
_NC = 2
_NS = 16
_NW = _NC * _NS
_NB = 512
_UNITS = _H * (_B // _NB)
_UPW = _UNITS // _NW
_L = 16


@functools.partial(
    pl.kernel,
    out_type=jax.ShapeDtypeStruct((_H, _D, _B), jnp.float32),
    mesh=plsc.VectorSubcoreMesh(core_axis_name="c", subcore_axis_name="s"),
    scratch_types=[
        pltpu.VMEM((_NB,), jnp.int32),
        pltpu.VMEM((_NB,), jnp.int32),
        pltpu.VMEM((_NB, 128), jnp.float32),
        pltpu.VMEM((_D, _NB), jnp.float32),
        pltpu.SemaphoreType.DMA,
    ],
    compiler_params=pltpu.CompilerParams(needs_layout_passes=False),
)
def _sc_embed(tbl_hbm, idx_hbm, out_hbm, idx_v, line_v, rows_v, trans_v, sem):
  wid = lax.axis_index("s") * _NC + lax.axis_index("c")

  def unit_body(u, carry):
    unit = wid * _UPW + u
    h = unit // (_B // _NB)
    b0 = (unit % (_B // _NB)) * _NB
    off = h * _B + b0
    pltpu.sync_copy(idx_hbm.at[pl.ds(off, _NB)], idx_v)

    def mk_lines(i, c):
      v = idx_v[pl.ds(i * _L, _L)]
      line_v[pl.ds(i * _L, _L)] = jax.lax.shift_right_logical(v, 2)
      return c

    lax.fori_loop(0, _NB // _L, mk_lines, 0)
    pltpu.async_copy(tbl_hbm.at[line_v], rows_v, sem).wait()

    def transpose_grp(i, c):
      v = idx_v[pl.ds(i * _L, _L)]
      colbase = jax.lax.shift_left(jax.lax.bitwise_and(v, 3), 5)
      rowvec = jax.lax.iota(jnp.int32, _L) + i * _L
      for d in range(_D):
        vals = plsc.load_gather(rows_v, [rowvec, colbase + d])
        trans_v[d, pl.ds(i * _L, _L)] = vals
      return c

    lax.fori_loop(0, _NB // _L, transpose_grp, 0)
    pltpu.sync_copy(trans_v, out_hbm.at[h, :, pl.ds(b0, _NB)])
    return carry

  lax.fori_loop(0, _UPW, unit_body, 0)


def kernel(inputs, embeddings):
  idx_hmaj = inputs.T.astype(jnp.int32).reshape(_H * _B)
  tbl = embeddings.reshape(_VOCAB // 4, 128)
  out3 = _sc_embed(tbl, idx_hmaj)
  return out3.transpose(2, 0, 1)


# speedup vs baseline: 1.6662x; 1.2364x over previous
import functools

import jax
import jax.numpy as jnp
from jax import lax
from jax.experimental import pallas as pl
from jax.experimental.pallas import tpu as pltpu
from jax.experimental.pallas import tpu_sc as plsc

_VOCAB = 1000000
_B = 16384
_H = 50
_D = 32
_NC = 2
_NS = 16
_NW = _NC * _NS
_NB = 256
_UNITS = _H * (_B // _NB)
_UPW = _UNITS // _NW
_UPAIRS = _UPW // 2
_L = 16
_BPW = _UPW * _NB


@functools.partial(
    pl.kernel,
    out_type=jax.ShapeDtypeStruct((_H, _D, _B), jnp.float32),
    mesh=plsc.VectorSubcoreMesh(core_axis_name="c", subcore_axis_name="s"),
    scratch_types=[
        pltpu.VMEM((_BPW,), jnp.int32),
        pltpu.VMEM((_NB, 128), jnp.float32),
        pltpu.VMEM((_NB, 128), jnp.float32),
        pltpu.VMEM((_D, _NB), jnp.float32),
        pltpu.VMEM((_D, _NB), jnp.float32),
        pltpu.SemaphoreType.DMA,
        pltpu.SemaphoreType.DMA,
        pltpu.SemaphoreType.DMA,
        pltpu.SemaphoreType.DMA,
        pltpu.SemaphoreType.DMA,
    ],
    compiler_params=pltpu.CompilerParams(needs_layout_passes=False),
)
def _sc_embed(tbl_hbm, idx_hbm, out_hbm, idx_v, r0, r1, t0, t1,
              g0, g1, s0, s1, isem):
  rows = (r0, r1)
  trans = (t0, t1)
  gsem = (g0, g1)
  ssem = (s0, s1)
  wid = lax.axis_index("s") * _NC + lax.axis_index("c")
  gu0 = wid * _UPW

  pltpu.async_copy(idx_hbm.at[pl.ds(gu0 * _NB, _BPW)], idx_v, isem).wait()

  def start_gather(u, b):
    pltpu.async_copy(tbl_hbm.at[idx_v.at[pl.ds(u * _NB, _NB)]],
                     rows[b], gsem[b])

  def wait_gather(b):
    pltpu.make_async_copy(
        tbl_hbm.at[idx_v.at[pl.ds(0, _NB)]], rows[b], gsem[b]).wait()

  def transpose_unit(b):
    def grp_body(i, c):
      rowvec = lax.iota(jnp.int32, _L) + i * _L
      for d in range(_D):
        dvec = jnp.full((_L,), d, jnp.int32)
        vals = plsc.load_gather(rows[b], [rowvec, dvec])
        trans[b][d, pl.ds(i * _L, _L)] = vals
      return c

    lax.fori_loop(0, _NB // _L, grp_body, 0)

  def start_store(u, b):
    gu = gu0 + u
    h = gu // (_B // _NB)
    b0 = (gu % (_B // _NB)) * _NB
    pltpu.async_copy(trans[b], out_hbm.at[h, :, pl.ds(b0, _NB)], ssem[b])

  def wait_store(b):
    pltpu.make_async_copy(
        trans[b], out_hbm.at[0, :, pl.ds(0, _NB)], ssem[b]).wait()

  start_gather(0, 0)

  def pair_body(g, carry):
    u0 = 2 * g
    u1 = u0 + 1
    wait_gather(0)
    start_gather(u1, 1)

    @pl.when(g > 0)
    def _():
      wait_store(0)

    transpose_unit(0)
    start_store(u0, 0)
    wait_gather(1)

    @pl.when(u1 + 1 < _UPW)
    def _():
      start_gather(u1 + 1, 0)

    @pl.when(g > 0)
    def _():
      wait_store(1)

    transpose_unit(1)
    start_store(u1, 1)
    return carry

  lax.fori_loop(0, _UPAIRS, pair_body, 0)
  wait_store(0)
  wait_store(1)


def kernel(inputs, embeddings):
  idx_hmaj = inputs.T.astype(jnp.int32).reshape(_H * _B)
  tbl = jnp.pad(embeddings, ((0, 0), (0, 128 - _D)))
  out3 = _sc_embed(tbl, idx_hmaj)
  return out3.transpose(2, 0, 1)
